# baseline (speedup 1.0000x reference)
import jax
import jax.numpy as jnp
from jax.experimental import pallas as pl
from jax.experimental.pallas import tpu as pltpu


def _slice_copy_body(seq_len):
    def body(w_ref, o_ref, sem):
        copy = pltpu.make_async_copy(w_ref.at[pl.ds(0, seq_len), :], o_ref, sem)
        copy.start()
        copy.wait()
    return body


def kernel(indices, weight):
    seq_len = indices.shape[-2]
    return pl.pallas_call(
        _slice_copy_body(seq_len),
        out_shape=jax.ShapeDtypeStruct((seq_len, weight.shape[1]), weight.dtype),
        in_specs=[pl.BlockSpec(memory_space=pl.ANY)],
        out_specs=pl.BlockSpec(memory_space=pl.ANY),
        scratch_shapes=[pltpu.SemaphoreType.DMA],
    )(weight)


# speedup vs baseline: 1.0014x; 1.0014x over previous
import jax
import jax.numpy as jnp
from jax.experimental import pallas as pl
from jax.experimental.pallas import tpu as pltpu

_NCHUNK = 8


def _slice_copy_body(seq_len, nchunk):
    rows = seq_len // nchunk

    def body(w_ref, o_ref, sem):
        copies = []
        for i in range(nchunk):
            sl = pl.ds(i * rows, rows)
            copies.append(pltpu.make_async_copy(w_ref.at[sl, :], o_ref.at[sl, :], sem))
        for c in copies:
            c.start()
        for c in copies:
            c.wait()

    return body


def kernel(indices, weight):
    seq_len = indices.shape[-2]
    nchunk = _NCHUNK
    while seq_len % nchunk:
        nchunk //= 2
    return pl.pallas_call(
        _slice_copy_body(seq_len, nchunk),
        out_shape=jax.ShapeDtypeStruct((seq_len, weight.shape[1]), weight.dtype),
        in_specs=[pl.BlockSpec(memory_space=pl.ANY)],
        out_specs=pl.BlockSpec(memory_space=pl.ANY),
        scratch_shapes=[pltpu.SemaphoreType.DMA],
    )(weight)


# speedup vs baseline: 27.9502x; 27.9120x over previous
import jax
import jax.numpy as jnp
from jax.experimental import pallas as pl
from jax.experimental.pallas import tpu as pltpu

_BLOCK_ROWS = 256


def _copy_body(w_ref, o_ref):
    o_ref[...] = w_ref[...]


def kernel(indices, weight):
    seq_len = indices.shape[-2]
    cols = weight.shape[1]
    br = min(_BLOCK_ROWS, seq_len)
    while seq_len % br:
        br //= 2
    grid = seq_len // br
    return pl.pallas_call(
        _copy_body,
        grid=(grid,),
        out_shape=jax.ShapeDtypeStruct((seq_len, cols), weight.dtype),
        in_specs=[pl.BlockSpec((br, cols), lambda i: (i, 0))],
        out_specs=pl.BlockSpec((br, cols), lambda i: (i, 0)),
    )(weight)


# speedup vs baseline: 34.3270x; 1.2281x over previous
import jax
import jax.numpy as jnp
from jax.experimental import pallas as pl
from jax.experimental.pallas import tpu as pltpu

_BLOCK_ROWS = 512


def _copy_body(w_ref, o_ref):
    o_ref[...] = w_ref[...]


def kernel(indices, weight):
    seq_len = indices.shape[-2]
    cols = weight.shape[1]
    br = min(_BLOCK_ROWS, seq_len)
    while seq_len % br:
        br //= 2
    grid = seq_len // br
    return pl.pallas_call(
        _copy_body,
        grid=(grid,),
        out_shape=jax.ShapeDtypeStruct((seq_len, cols), weight.dtype),
        in_specs=[pl.BlockSpec((br, cols), lambda i: (i, 0))],
        out_specs=pl.BlockSpec((br, cols), lambda i: (i, 0)),
    )(weight)


# speedup vs baseline: 42.5995x; 1.2410x over previous
import jax
import jax.numpy as jnp
from jax.experimental import pallas as pl
from jax.experimental.pallas import tpu as pltpu

_BLOCK_ROWS = 1024


def _copy_body(w_ref, o_ref):
    o_ref[...] = w_ref[...]


def kernel(indices, weight):
    seq_len = indices.shape[-2]
    cols = weight.shape[1]
    br = min(_BLOCK_ROWS, seq_len)
    while seq_len % br:
        br //= 2
    grid = seq_len // br
    return pl.pallas_call(
        _copy_body,
        grid=(grid,),
        out_shape=jax.ShapeDtypeStruct((seq_len, cols), weight.dtype),
        in_specs=[pl.BlockSpec((br, cols), lambda i: (i, 0))],
        out_specs=pl.BlockSpec((br, cols), lambda i: (i, 0)),
    )(weight)
